# Initial kernel scaffold; baseline (speedup 1.0000x reference)
import jax
import jax.numpy as jnp
from jax.experimental import pallas as pl


def kernel(node_fea, edge_sh, edge_fea, edge_length_embedded, edge_index, W_pre, W_fc1, W_fc2, W_post):
    raise NotImplementedError("write your pallas kernel here")


# speedup vs baseline: 1.4613x; 1.4613x over previous
import functools
import math

import jax
import jax.numpy as jnp
import numpy as np
from jax import lax
from jax.experimental import pallas as pl
from jax.experimental.pallas import tpu as pltpu
from jax.experimental.pallas import tpu_sc as plsc

N = 10000
E = 160000
DN = 16
DE = 8
FC_IN = 16
HID = 16
IN1 = 2 * DN + DE

NC, NS = 2, 16
NW = NC * NS
EPW = E // NW
CH = 128
NCH = -(-EPW // CH)
EBUF = NCH * CH
PADW = EBUF - EPW

BE = 2000


@functools.cache
def _mesh():
    return plsc.VectorSubcoreMesh(
        core_axis_name="c", subcore_axis_name="s", num_cores=NC, num_subcores=NS
    )


def _gather_body(x_hbm, ii_hbm, ij_hbm, xi_hbm, xj_hbm, idxi_v, idxj_v, rows_v, sem):
    wid = lax.axis_index("s") * NC + lax.axis_index("c")
    base = wid * EPW
    pltpu.sync_copy(ii_hbm.at[wid], idxi_v)
    pltpu.sync_copy(ij_hbm.at[wid], idxj_v)

    def body_i(j, carry):
        pltpu.async_copy(
            x_hbm.at[idxi_v.at[j]], rows_v.at[pl.ds(j * CH, CH)], sem
        ).wait()
        return carry

    lax.fori_loop(0, NCH, body_i, 0)
    pltpu.sync_copy(rows_v.at[pl.ds(0, EPW)], xi_hbm.at[pl.ds(base, EPW)])

    def body_j(j, carry):
        pltpu.async_copy(
            x_hbm.at[idxj_v.at[j]], rows_v.at[pl.ds(j * CH, CH)], sem
        ).wait()
        return carry

    lax.fori_loop(0, NCH, body_j, 0)
    pltpu.sync_copy(rows_v.at[pl.ds(0, EPW)], xj_hbm.at[pl.ds(base, EPW)])


@functools.cache
def _gather_k():
    return pl.kernel(
        _gather_body,
        out_type=(
            jax.ShapeDtypeStruct((E, DN), jnp.float32),
            jax.ShapeDtypeStruct((E, DN), jnp.float32),
        ),
        mesh=_mesh(),
        scratch_types=[
            pltpu.VMEM((NCH, CH), jnp.int32),
            pltpu.VMEM((NCH, CH), jnp.int32),
            pltpu.VMEM((EBUF, DN), jnp.float32),
            pltpu.SemaphoreType.DMA,
        ],
        compiler_params=pltpu.CompilerParams(use_tc_tiling_on_sc=False),
    )


def _scatter_body(eu_hbm, ii_hbm, zero_hbm, out_hbm, idx_v, upd_v, agg_sh):
    c = lax.axis_index("c")
    s = lax.axis_index("s")
    wid = s * NC + c
    base = wid * EPW

    @pl.when(s == 0)
    def _zero():
        pltpu.sync_copy(zero_hbm, agg_sh)

    pltpu.sync_copy(ii_hbm.at[wid], idx_v)
    pltpu.sync_copy(eu_hbm.at[pl.ds(base, EPW)], upd_v.at[pl.ds(0, EPW)])
    pltpu.sync_copy(zero_hbm.at[pl.ds(0, PADW)], upd_v.at[pl.ds(EPW, PADW)])
    plsc.subcore_barrier()

    def body(j, carry):
        pltpu.sync_copy(
            upd_v.at[pl.ds(j * CH, CH)], agg_sh.at[idx_v.at[j]], add=True
        )
        return carry

    lax.fori_loop(0, NCH, body, 0)
    plsc.subcore_barrier()
    rows = N // NS
    pltpu.sync_copy(
        agg_sh.at[pl.ds(s * rows, rows)], out_hbm.at[c, pl.ds(s * rows, rows)]
    )


@functools.cache
def _scatter_k():
    return pl.kernel(
        _scatter_body,
        out_type=jax.ShapeDtypeStruct((NC, N, DN), jnp.float32),
        mesh=_mesh(),
        scratch_types=[
            pltpu.VMEM((NCH, CH), jnp.int32),
            pltpu.VMEM((EBUF, DN), jnp.float32),
            pltpu.VMEM_SHARED((N, DN), jnp.float32),
        ],
        compiler_params=pltpu.CompilerParams(use_tc_tiling_on_sc=False),
    )


def _dense_body(xi_ref, xj_ref, ef_ref, ele_ref, sh_ref,
                wci_ref, wcj_ref, wcf_ref, w1_ref, out_ref):
    ele = ele_ref[...]
    pre = jnp.dot(ele, w1_ref[...], preferred_element_type=jnp.float32)
    h = pre * jax.nn.sigmoid(pre)
    y = (
        jnp.dot(xi_ref[...], wci_ref[...], preferred_element_type=jnp.float32)
        + jnp.dot(xj_ref[...], wcj_ref[...], preferred_element_type=jnp.float32)
        + jnp.dot(ef_ref[...], wcf_ref[...], preferred_element_type=jnp.float32)
    )
    acc = h[:, 0:1] * y[:, 0:DN]
    for m in range(1, HID):
        acc = acc + h[:, m : m + 1] * y[:, m * DN : (m + 1) * DN]
    out_ref[...] = acc * sh_ref[...]


def _dense(xi, xj, ef, ele, sh, wci, wcj, wcf, w1):
    grid = E // BE
    full = lambda shape: pl.BlockSpec(shape, lambda i: (0, 0))
    return pl.pallas_call(
        _dense_body,
        grid=(grid,),
        in_specs=[
            pl.BlockSpec((BE, DN), lambda i: (i, 0)),
            pl.BlockSpec((BE, DN), lambda i: (i, 0)),
            pl.BlockSpec((BE, DE), lambda i: (i, 0)),
            pl.BlockSpec((BE, FC_IN), lambda i: (i, 0)),
            pl.BlockSpec((BE, 1), lambda i: (i, 0)),
            full((DN, HID * DN)),
            full((DN, HID * DN)),
            full((DE, HID * DN)),
            full((FC_IN, HID)),
        ],
        out_specs=pl.BlockSpec((BE, DN), lambda i: (i, 0)),
        out_shape=jax.ShapeDtypeStruct((E, DN), jnp.float32),
    )(xi, xj, ef, ele, sh, wci, wcj, wcf, w1)


def _final_body(p_ref, w_ref, out_ref):
    agg = p_ref[0] + p_ref[1]
    z = jnp.dot(agg, w_ref[...], preferred_element_type=jnp.float32)
    out_ref[...] = z * jax.nn.sigmoid(z)


def _final(parts, w_post):
    return pl.pallas_call(
        _final_body,
        out_shape=jax.ShapeDtypeStruct((N, DN), jnp.float32),
    )(parts, w_post)


def kernel(node_fea, edge_sh, edge_fea, edge_length_embedded, edge_index,
           W_pre, W_fc1, W_fc2, W_post):
    ii = edge_index[0]
    ij = edge_index[1]
    def chunked(idx):
        return jnp.pad(idx.reshape(NW, EPW), ((0, 0), (0, PADW))).reshape(
            NW, NCH, CH
        )

    ii3 = chunked(ii)
    ij3 = chunked(ij)

    w1n = W_fc1 / np.sqrt(FC_IN)
    wpn = W_pre / np.sqrt(DN)
    wcat = (
        (W_fc2 / np.sqrt(HID))
        .reshape(HID, IN1, DN)
        .transpose(1, 0, 2)
        .reshape(IN1, HID * DN)
    ) / np.sqrt(IN1 * 1)
    wci = wpn @ wcat[0:DN]
    wcj = wpn @ wcat[DN : 2 * DN]
    wcf = wcat[2 * DN :]
    wpostn = W_post / np.sqrt(DN)

    zeros = jnp.zeros((N, DN), jnp.float32)

    xi_raw, xj_raw = _gather_k()(node_fea, ii3, ij3)
    eu = _dense(xi_raw, xj_raw, edge_fea, edge_length_embedded, edge_sh,
                wci, wcj, wcf, w1n)
    parts = _scatter_k()(eu, ii3, zeros)
    return _final(parts, wpostn)


# speedup vs baseline: 2.8766x; 1.9685x over previous
import functools
import math

import jax
import jax.numpy as jnp
import numpy as np
from jax import lax
from jax.experimental import pallas as pl
from jax.experimental.pallas import tpu as pltpu
from jax.experimental.pallas import tpu_sc as plsc

N = 10000
E = 160000
DN = 16
DE = 8
FC_IN = 16
HID = 16
IN1 = 2 * DN + DE

NC, NS = 2, 16
NW = NC * NS
EPW = E // NW
CH = 128
NCH = -(-EPW // CH)
EBUF = NCH * CH
PADW = EBUF - EPW

BE = 2000


@functools.cache
def _mesh():
    return plsc.VectorSubcoreMesh(
        core_axis_name="c", subcore_axis_name="s", num_cores=NC, num_subcores=NS
    )


def _gather_body(x_hbm, ii_hbm, ij_hbm, xi_hbm, xj_hbm, idxi_v, idxj_v, rows_v, sem):
    wid = lax.axis_index("s") * NC + lax.axis_index("c")
    base = wid * EPW
    pltpu.sync_copy(ii_hbm.at[wid], idxi_v)
    pltpu.sync_copy(ij_hbm.at[wid], idxj_v)

    def body_i(j, carry):
        pltpu.async_copy(
            x_hbm.at[idxi_v.at[j]], rows_v.at[pl.ds(j * CH, CH)], sem
        ).wait()
        return carry

    lax.fori_loop(0, NCH, body_i, 0)
    pltpu.sync_copy(rows_v.at[pl.ds(0, EPW)], xi_hbm.at[pl.ds(base, EPW)])

    def body_j(j, carry):
        pltpu.async_copy(
            x_hbm.at[idxj_v.at[j]], rows_v.at[pl.ds(j * CH, CH)], sem
        ).wait()
        return carry

    lax.fori_loop(0, NCH, body_j, 0)
    pltpu.sync_copy(rows_v.at[pl.ds(0, EPW)], xj_hbm.at[pl.ds(base, EPW)])


@functools.cache
def _gather_k():
    return pl.kernel(
        _gather_body,
        out_type=(
            jax.ShapeDtypeStruct((E, DN), jnp.float32),
            jax.ShapeDtypeStruct((E, DN), jnp.float32),
        ),
        mesh=_mesh(),
        scratch_types=[
            pltpu.VMEM((NCH, CH), jnp.int32),
            pltpu.VMEM((NCH, CH), jnp.int32),
            pltpu.VMEM((EBUF, DN), jnp.float32),
            pltpu.SemaphoreType.DMA,
        ],
        compiler_params=pltpu.CompilerParams(use_tc_tiling_on_sc=False),
    )


def _scatter_body(eu_hbm, ii_hbm, zero_hbm, out_hbm, idx_v, upd_v, agg_sh):
    c = lax.axis_index("c")
    s = lax.axis_index("s")
    wid = s * NC + c
    base = wid * EPW

    @pl.when(s == 0)
    def _zero():
        pltpu.sync_copy(zero_hbm, agg_sh)

    pltpu.sync_copy(ii_hbm.at[wid], idx_v)
    pltpu.sync_copy(eu_hbm.at[pl.ds(base, EPW)], upd_v.at[pl.ds(0, EPW)])
    pltpu.sync_copy(zero_hbm.at[pl.ds(0, PADW)], upd_v.at[pl.ds(EPW, PADW)])
    plsc.subcore_barrier()

    def body(j, carry):
        pltpu.sync_copy(
            upd_v.at[pl.ds(j * CH, CH)], agg_sh.at[idx_v.at[j]], add=True
        )
        return carry

    lax.fori_loop(0, NCH, body, 0)
    plsc.subcore_barrier()
    rows = N // NS
    pltpu.sync_copy(
        agg_sh.at[pl.ds(s * rows, rows)], out_hbm.at[c, pl.ds(s * rows, rows)]
    )


@functools.cache
def _scatter_k():
    return pl.kernel(
        _scatter_body,
        out_type=jax.ShapeDtypeStruct((NC, N, DN), jnp.float32),
        mesh=_mesh(),
        scratch_types=[
            pltpu.VMEM((NCH, CH), jnp.int32),
            pltpu.VMEM((EBUF, DN), jnp.float32),
            pltpu.VMEM_SHARED((N, DN), jnp.float32),
        ],
        compiler_params=pltpu.CompilerParams(use_tc_tiling_on_sc=False),
    )


@functools.cache
def _sel_r():
    r = np.zeros((HID, HID * DN), np.float32)
    for m in range(HID):
        r[m, m * DN : (m + 1) * DN] = 1.0
    return jnp.asarray(r)


@functools.cache
def _sel_s():
    s = np.zeros((HID * DN, DN), np.float32)
    for m in range(HID):
        for k in range(DN):
            s[m * DN + k, k] = 1.0
    return jnp.asarray(s)


def _dense_body(xi_ref, xj_ref, ef_ref, ele_ref, sh_ref,
                wci_ref, wcj_ref, wcf_ref, w1_ref, r_ref, s_ref, out_ref):
    ele = ele_ref[...]
    pre = jnp.dot(ele, w1_ref[...], preferred_element_type=jnp.float32)
    h = pre * jax.nn.sigmoid(pre) * sh_ref[...]
    y = (
        jnp.dot(xi_ref[...], wci_ref[...], preferred_element_type=jnp.float32)
        + jnp.dot(xj_ref[...], wcj_ref[...], preferred_element_type=jnp.float32)
        + jnp.dot(ef_ref[...], wcf_ref[...], preferred_element_type=jnp.float32)
    )
    hx = jnp.dot(h, r_ref[...], preferred_element_type=jnp.float32)
    out_ref[...] = jnp.dot(y * hx, s_ref[...], preferred_element_type=jnp.float32)


def _dense(xi, xj, ef, ele, sh, wci, wcj, wcf, w1):
    grid = E // BE
    full = lambda shape: pl.BlockSpec(shape, lambda i: (0, 0))
    return pl.pallas_call(
        _dense_body,
        grid=(grid,),
        in_specs=[
            pl.BlockSpec((BE, DN), lambda i: (i, 0)),
            pl.BlockSpec((BE, DN), lambda i: (i, 0)),
            pl.BlockSpec((BE, DE), lambda i: (i, 0)),
            pl.BlockSpec((BE, FC_IN), lambda i: (i, 0)),
            pl.BlockSpec((BE, 1), lambda i: (i, 0)),
            full((DN, HID * DN)),
            full((DN, HID * DN)),
            full((DE, HID * DN)),
            full((FC_IN, HID)),
            full((HID, HID * DN)),
            full((HID * DN, DN)),
        ],
        out_specs=pl.BlockSpec((BE, DN), lambda i: (i, 0)),
        out_shape=jax.ShapeDtypeStruct((E, DN), jnp.float32),
    )(xi, xj, ef, ele, sh, wci, wcj, wcf, w1, _sel_r(), _sel_s())


def _final_body(p_ref, w_ref, out_ref):
    agg = p_ref[0] + p_ref[1]
    z = jnp.dot(agg, w_ref[...], preferred_element_type=jnp.float32)
    out_ref[...] = z * jax.nn.sigmoid(z)


def _final(parts, w_post):
    return pl.pallas_call(
        _final_body,
        out_shape=jax.ShapeDtypeStruct((N, DN), jnp.float32),
    )(parts, w_post)


def kernel(node_fea, edge_sh, edge_fea, edge_length_embedded, edge_index,
           W_pre, W_fc1, W_fc2, W_post):
    ii = edge_index[0]
    ij = edge_index[1]
    def chunked(idx):
        return jnp.pad(idx.reshape(NW, EPW), ((0, 0), (0, PADW))).reshape(
            NW, NCH, CH
        )

    ii3 = chunked(ii)
    ij3 = chunked(ij)

    w1n = W_fc1 / np.sqrt(FC_IN)
    wpn = W_pre / np.sqrt(DN)
    wcat = (
        (W_fc2 / np.sqrt(HID))
        .reshape(HID, IN1, DN)
        .transpose(1, 0, 2)
        .reshape(IN1, HID * DN)
    ) / np.sqrt(IN1 * 1)
    wci = wpn @ wcat[0:DN]
    wcj = wpn @ wcat[DN : 2 * DN]
    wcf = wcat[2 * DN :]
    wpostn = W_post / np.sqrt(DN)

    zeros = jnp.zeros((N, DN), jnp.float32)

    xi_raw, xj_raw = _gather_k()(node_fea, ii3, ij3)
    eu = _dense(xi_raw, xj_raw, edge_fea, edge_length_embedded, edge_sh,
                wci, wcj, wcf, w1n)
    parts = _scatter_k()(eu, ii3, zeros)
    return _final(parts, wpostn)


# speedup vs baseline: 3.0540x; 1.0617x over previous
import functools
import math

import jax
import jax.numpy as jnp
import numpy as np
from jax import lax
from jax.experimental import pallas as pl
from jax.experimental.pallas import tpu as pltpu
from jax.experimental.pallas import tpu_sc as plsc

N = 10000
E = 160000
DN = 16
DE = 8
FC_IN = 16
HID = 16
IN1 = 2 * DN + DE

NC, NS = 2, 16
NW = NC * NS
EPW = E // NW
CH = 128
NCH = -(-EPW // CH)
NFULL = EPW // CH
TAIL = EPW - NFULL * CH
EBUF = NCH * CH

BE = 2000


@functools.cache
def _mesh():
    return plsc.VectorSubcoreMesh(
        core_axis_name="c", subcore_axis_name="s", num_cores=NC, num_subcores=NS
    )


def _gather_body(x_hbm, ii_hbm, ij_hbm, xi_hbm, xj_hbm, idx_v, rows_v, sem):
    wid = lax.axis_index("s") * NC + lax.axis_index("c")
    base = wid * EPW

    for idx_hbm, out_hbm in ((ii_hbm, xi_hbm), (ij_hbm, xj_hbm)):
        pltpu.sync_copy(idx_hbm.at[pl.ds(base, EPW)], idx_v)

        def body(j, carry):
            pltpu.async_copy(
                x_hbm.at[idx_v.at[pl.ds(j * CH, CH)]],
                rows_v.at[pl.ds(j * CH, CH)],
                sem,
            ).wait()
            return carry

        lax.fori_loop(0, NFULL, body, 0)
        pltpu.async_copy(
            x_hbm.at[idx_v.at[pl.ds(NFULL * CH, TAIL)]],
            rows_v.at[pl.ds(NFULL * CH, TAIL)],
            sem,
        ).wait()
        pltpu.sync_copy(rows_v, out_hbm.at[pl.ds(base, EPW)])


@functools.cache
def _gather_k():
    return pl.kernel(
        _gather_body,
        out_type=(
            jax.ShapeDtypeStruct((E, DN), jnp.float32),
            jax.ShapeDtypeStruct((E, DN), jnp.float32),
        ),
        mesh=_mesh(),
        scratch_types=[
            pltpu.VMEM((EPW,), jnp.int32),
            pltpu.VMEM((EPW, DN), jnp.float32),
            pltpu.SemaphoreType.DMA,
        ],
        compiler_params=pltpu.CompilerParams(use_tc_tiling_on_sc=False),
    )


def _scatter_body(eu_hbm, ii_hbm, zero_hbm, out_hbm, idx_v, upd_v, agg_sh, sem):
    c = lax.axis_index("c")
    s = lax.axis_index("s")
    wid = s * NC + c
    base = wid * EPW

    @pl.when(s == 0)
    def _zero():
        pltpu.sync_copy(zero_hbm, agg_sh)

    def fire(j, carry):
        pltpu.async_copy(
            ii_hbm.at[pl.ds(base + j * CH, CH)], idx_v.at[j], sem
        )
        return carry

    lax.fori_loop(0, NFULL, fire, 0)

    def drain(j, carry):
        pltpu.make_async_copy(
            ii_hbm.at[pl.ds(base, CH)], idx_v.at[0], sem
        ).wait()
        return carry

    lax.fori_loop(0, NFULL, drain, 0)
    pltpu.sync_copy(ii_hbm.at[pl.ds(base, CH)], idx_v.at[NFULL])
    pltpu.sync_copy(
        ii_hbm.at[pl.ds(base + NFULL * CH, TAIL)],
        idx_v.at[NFULL, pl.ds(0, TAIL)],
    )
    pltpu.sync_copy(eu_hbm.at[pl.ds(base, EPW)], upd_v.at[pl.ds(0, EPW)])
    pltpu.sync_copy(
        zero_hbm.at[pl.ds(0, EBUF - EPW)], upd_v.at[pl.ds(EPW, EBUF - EPW)]
    )
    plsc.subcore_barrier()

    def body(j, carry):
        pltpu.sync_copy(
            upd_v.at[pl.ds(j * CH, CH)], agg_sh.at[idx_v.at[j]], add=True
        )
        return carry

    lax.fori_loop(0, NCH, body, 0)
    plsc.subcore_barrier()
    rows = N // NS
    pltpu.sync_copy(
        agg_sh.at[pl.ds(s * rows, rows)], out_hbm.at[c, pl.ds(s * rows, rows)]
    )


@functools.cache
def _scatter_k():
    return pl.kernel(
        _scatter_body,
        out_type=jax.ShapeDtypeStruct((NC, N, DN), jnp.float32),
        mesh=_mesh(),
        scratch_types=[
            pltpu.VMEM((NCH, CH), jnp.int32),
            pltpu.VMEM((EBUF, DN), jnp.float32),
            pltpu.VMEM_SHARED((N, DN), jnp.float32),
            pltpu.SemaphoreType.DMA,
        ],
        compiler_params=pltpu.CompilerParams(use_tc_tiling_on_sc=False),
    )


@functools.cache
def _sel_r():
    r = np.zeros((HID, HID * DN), np.float32)
    for m in range(HID):
        r[m, m * DN : (m + 1) * DN] = 1.0
    return jnp.asarray(r)


@functools.cache
def _sel_s():
    s = np.zeros((HID * DN, DN), np.float32)
    for m in range(HID):
        for k in range(DN):
            s[m * DN + k, k] = 1.0
    return jnp.asarray(s)


def _dense_body(xi_ref, xj_ref, ef_ref, ele_ref, sh_ref,
                wci_ref, wcj_ref, wcf_ref, w1_ref, r_ref, s_ref, out_ref):
    ele = ele_ref[...]
    pre = jnp.dot(ele, w1_ref[...], preferred_element_type=jnp.float32)
    h = pre * jax.nn.sigmoid(pre) * sh_ref[...]
    y = (
        jnp.dot(xi_ref[...], wci_ref[...], preferred_element_type=jnp.float32)
        + jnp.dot(xj_ref[...], wcj_ref[...], preferred_element_type=jnp.float32)
        + jnp.dot(ef_ref[...], wcf_ref[...], preferred_element_type=jnp.float32)
    )
    hx = jnp.dot(h, r_ref[...], preferred_element_type=jnp.float32)
    out_ref[...] = jnp.dot(y * hx, s_ref[...], preferred_element_type=jnp.float32)


def _dense(xi, xj, ef, ele, sh, wci, wcj, wcf, w1):
    grid = E // BE
    full = lambda shape: pl.BlockSpec(shape, lambda i: (0, 0))
    return pl.pallas_call(
        _dense_body,
        grid=(grid,),
        in_specs=[
            pl.BlockSpec((BE, DN), lambda i: (i, 0)),
            pl.BlockSpec((BE, DN), lambda i: (i, 0)),
            pl.BlockSpec((BE, DE), lambda i: (i, 0)),
            pl.BlockSpec((BE, FC_IN), lambda i: (i, 0)),
            pl.BlockSpec((BE, 1), lambda i: (i, 0)),
            full((DN, HID * DN)),
            full((DN, HID * DN)),
            full((DE, HID * DN)),
            full((FC_IN, HID)),
            full((HID, HID * DN)),
            full((HID * DN, DN)),
        ],
        out_specs=pl.BlockSpec((BE, DN), lambda i: (i, 0)),
        out_shape=jax.ShapeDtypeStruct((E, DN), jnp.float32),
    )(xi, xj, ef, ele, sh, wci, wcj, wcf, w1, _sel_r(), _sel_s())


def _final_body(p_ref, w_ref, out_ref):
    agg = p_ref[0] + p_ref[1]
    z = jnp.dot(agg, w_ref[...], preferred_element_type=jnp.float32)
    out_ref[...] = z * jax.nn.sigmoid(z)


def _final(parts, w_post):
    return pl.pallas_call(
        _final_body,
        out_shape=jax.ShapeDtypeStruct((N, DN), jnp.float32),
    )(parts, w_post)


def kernel(node_fea, edge_sh, edge_fea, edge_length_embedded, edge_index,
           W_pre, W_fc1, W_fc2, W_post):
    ii = edge_index[0]
    ij = edge_index[1]

    w1n = W_fc1 / np.sqrt(FC_IN)
    wpn = W_pre / np.sqrt(DN)
    wcat = (
        (W_fc2 / np.sqrt(HID))
        .reshape(HID, IN1, DN)
        .transpose(1, 0, 2)
        .reshape(IN1, HID * DN)
    ) / np.sqrt(IN1 * 1)
    wci = wpn @ wcat[0:DN]
    wcj = wpn @ wcat[DN : 2 * DN]
    wcf = wcat[2 * DN :]
    wpostn = W_post / np.sqrt(DN)

    zeros = jnp.zeros((N, DN), jnp.float32)

    xi_raw, xj_raw = _gather_k()(node_fea, ii, ij)
    eu = _dense(xi_raw, xj_raw, edge_fea, edge_length_embedded, edge_sh,
                wci, wcj, wcf, w1n)
    parts = _scatter_k()(eu, ii, zeros)
    return _final(parts, wpostn)


# speedup vs baseline: 3.7636x; 1.2323x over previous
import functools
import math

import jax
import jax.numpy as jnp
import numpy as np
from jax import lax
from jax.experimental import pallas as pl
from jax.experimental.pallas import tpu as pltpu
from jax.experimental.pallas import tpu_sc as plsc

N = 10000
E = 160000
DN = 16
DE = 8
FC_IN = 16
HID = 16
IN1 = 2 * DN + DE

NC, NS = 2, 16
NW = NC * NS
EPW = E // NW
CH = 128
NCH = -(-EPW // CH)
NFULL = EPW // CH
TAIL = EPW - NFULL * CH
EBUF = NCH * CH

BE = 3200


@functools.cache
def _mesh():
    return plsc.VectorSubcoreMesh(
        core_axis_name="c", subcore_axis_name="s", num_cores=NC, num_subcores=NS
    )


def _gather_body(x_hbm, ii_hbm, ij_hbm, xi_hbm, xj_hbm, idx_v, rows_v, sem):
    wid = lax.axis_index("s") * NC + lax.axis_index("c")
    base = wid * EPW

    for idx_hbm, out_hbm in ((ii_hbm, xi_hbm), (ij_hbm, xj_hbm)):
        pltpu.sync_copy(idx_hbm.at[pl.ds(base, EPW)], idx_v)

        def body(j, carry):
            pltpu.async_copy(
                x_hbm.at[idx_v.at[pl.ds(j * CH, CH)]],
                rows_v.at[pl.ds(j * CH, CH)],
                sem,
            ).wait()
            return carry

        lax.fori_loop(0, NFULL, body, 0)
        pltpu.async_copy(
            x_hbm.at[idx_v.at[pl.ds(NFULL * CH, TAIL)]],
            rows_v.at[pl.ds(NFULL * CH, TAIL)],
            sem,
        ).wait()
        pltpu.sync_copy(rows_v, out_hbm.at[pl.ds(base, EPW)])


@functools.cache
def _gather_k():
    return pl.kernel(
        _gather_body,
        out_type=(
            jax.ShapeDtypeStruct((E, DN), jnp.float32),
            jax.ShapeDtypeStruct((E, DN), jnp.float32),
        ),
        mesh=_mesh(),
        scratch_types=[
            pltpu.VMEM((EPW,), jnp.int32),
            pltpu.VMEM((EPW, DN), jnp.float32),
            pltpu.SemaphoreType.DMA,
        ],
        compiler_params=pltpu.CompilerParams(use_tc_tiling_on_sc=False),
    )


def _scatter_body(eu_hbm, ii_hbm, zero_hbm, out_hbm, idx_v, upd_v, agg_sh, sem):
    c = lax.axis_index("c")
    s = lax.axis_index("s")
    wid = s * NC + c
    base = wid * EPW

    @pl.when(s == 0)
    def _zero():
        pltpu.sync_copy(zero_hbm, agg_sh)

    def fire(j, carry):
        pltpu.async_copy(
            ii_hbm.at[pl.ds(base + j * CH, CH)], idx_v.at[j], sem
        )
        return carry

    lax.fori_loop(0, NFULL, fire, 0)

    def drain(j, carry):
        pltpu.make_async_copy(
            ii_hbm.at[pl.ds(base, CH)], idx_v.at[0], sem
        ).wait()
        return carry

    lax.fori_loop(0, NFULL, drain, 0)
    pltpu.sync_copy(ii_hbm.at[pl.ds(base, CH)], idx_v.at[NFULL])
    pltpu.sync_copy(
        ii_hbm.at[pl.ds(base + NFULL * CH, TAIL)],
        idx_v.at[NFULL, pl.ds(0, TAIL)],
    )
    pltpu.sync_copy(eu_hbm.at[pl.ds(base, EPW)], upd_v.at[pl.ds(0, EPW)])
    pltpu.sync_copy(
        zero_hbm.at[pl.ds(0, EBUF - EPW)], upd_v.at[pl.ds(EPW, EBUF - EPW)]
    )
    plsc.subcore_barrier()

    def body(j, carry):
        pltpu.sync_copy(
            upd_v.at[pl.ds(j * CH, CH)], agg_sh.at[idx_v.at[j]], add=True
        )
        return carry

    lax.fori_loop(0, NCH, body, 0)
    plsc.subcore_barrier()
    rows = N // NS
    pltpu.sync_copy(
        agg_sh.at[pl.ds(s * rows, rows)], out_hbm.at[c, pl.ds(s * rows, rows)]
    )


@functools.cache
def _scatter_k():
    return pl.kernel(
        _scatter_body,
        out_type=jax.ShapeDtypeStruct((NC, N, DN), jnp.float32),
        mesh=_mesh(),
        scratch_types=[
            pltpu.VMEM((NCH, CH), jnp.int32),
            pltpu.VMEM((EBUF, DN), jnp.float32),
            pltpu.VMEM_SHARED((N, DN), jnp.float32),
            pltpu.SemaphoreType.DMA,
        ],
        compiler_params=pltpu.CompilerParams(use_tc_tiling_on_sc=False),
    )


@functools.cache
def _sel_r():
    r = np.zeros((HID, HID * DN), np.float32)
    for m in range(HID):
        r[m, m * DN : (m + 1) * DN] = 1.0
    return jnp.asarray(r)


@functools.cache
def _sel_s():
    s = np.zeros((HID * DN, DN), np.float32)
    for m in range(HID):
        for k in range(DN):
            s[m * DN + k, k] = 1.0
    return jnp.asarray(s)


def _dense_body(xi_ref, xj_ref, eft_ref, elet_ref, sht_ref,
                wci_ref, wcj_ref, wcf_ref, w1_ref, r_ref, s_ref, out_ref):
    ele = elet_ref[...].T
    ef = eft_ref[...].T
    sh = sht_ref[...].T
    pre = jnp.dot(ele, w1_ref[...], preferred_element_type=jnp.float32)
    h = pre * jax.nn.sigmoid(pre) * sh
    y = (
        jnp.dot(xi_ref[...], wci_ref[...], preferred_element_type=jnp.float32)
        + jnp.dot(xj_ref[...], wcj_ref[...], preferred_element_type=jnp.float32)
        + jnp.dot(ef, wcf_ref[...], preferred_element_type=jnp.float32)
    )
    hx = jnp.dot(h, r_ref[...], preferred_element_type=jnp.float32)
    out_ref[...] = jnp.dot(y * hx, s_ref[...], preferred_element_type=jnp.float32)


def _dense(xi, xj, eft, elet, sht, wci, wcj, wcf, w1):
    grid = E // BE
    full = lambda shape: pl.BlockSpec(shape, lambda i: (0, 0))
    return pl.pallas_call(
        _dense_body,
        grid=(grid,),
        in_specs=[
            pl.BlockSpec((BE, DN), lambda i: (i, 0)),
            pl.BlockSpec((BE, DN), lambda i: (i, 0)),
            pl.BlockSpec((DE, BE), lambda i: (0, i)),
            pl.BlockSpec((FC_IN, BE), lambda i: (0, i)),
            pl.BlockSpec((1, BE), lambda i: (0, i)),
            full((DN, HID * DN)),
            full((DN, HID * DN)),
            full((DE, HID * DN)),
            full((FC_IN, HID)),
            full((HID, HID * DN)),
            full((HID * DN, DN)),
        ],
        out_specs=pl.BlockSpec((BE, DN), lambda i: (i, 0)),
        out_shape=jax.ShapeDtypeStruct((E, DN), jnp.float32),
    )(xi, xj, eft, elet, sht, wci, wcj, wcf, w1, _sel_r(), _sel_s())


def _final_body(p_ref, w_ref, out_ref):
    agg = p_ref[0] + p_ref[1]
    z = jnp.dot(agg, w_ref[...], preferred_element_type=jnp.float32)
    out_ref[...] = z * jax.nn.sigmoid(z)


def _final(parts, w_post):
    return pl.pallas_call(
        _final_body,
        out_shape=jax.ShapeDtypeStruct((N, DN), jnp.float32),
    )(parts, w_post)


def kernel(node_fea, edge_sh, edge_fea, edge_length_embedded, edge_index,
           W_pre, W_fc1, W_fc2, W_post):
    ii = edge_index[0]
    ij = edge_index[1]

    w1n = W_fc1 / np.sqrt(FC_IN)
    wpn = W_pre / np.sqrt(DN)
    wcat = (
        (W_fc2 / np.sqrt(HID))
        .reshape(HID, IN1, DN)
        .transpose(1, 0, 2)
        .reshape(IN1, HID * DN)
    ) / np.sqrt(IN1 * 1)
    wci = wpn @ wcat[0:DN]
    wcj = wpn @ wcat[DN : 2 * DN]
    wcf = wcat[2 * DN :]
    wpostn = W_post / np.sqrt(DN)

    zeros = jnp.zeros((N, DN), jnp.float32)

    xi_raw, xj_raw = _gather_k()(node_fea, ii, ij)
    eu = _dense(xi_raw, xj_raw, edge_fea.T, edge_length_embedded.T, edge_sh.T,
                wci, wcj, wcf, w1n)
    parts = _scatter_k()(eu, ii, zeros)
    return _final(parts, wpostn)


# speedup vs baseline: 4.2323x; 1.1245x over previous
import functools
import math

import jax
import jax.numpy as jnp
import numpy as np
from jax import lax
from jax.experimental import pallas as pl
from jax.experimental.pallas import tpu as pltpu
from jax.experimental.pallas import tpu_sc as plsc

N = 10000
E = 160000
DN = 16
DE = 8
FC_IN = 16
HID = 16
IN1 = 2 * DN + DE

NC, NS = 2, 16
NW = NC * NS
EPW = E // NW
CH = 128
NCH = -(-EPW // CH)
NFULL = EPW // CH
TAIL = EPW - NFULL * CH
EBUF = NCH * CH

BE = 3200


@functools.cache
def _mesh():
    return plsc.VectorSubcoreMesh(
        core_axis_name="c", subcore_axis_name="s", num_cores=NC, num_subcores=NS
    )


def _gather_body(x_hbm, ii_hbm, ij_hbm, xi_hbm, xj_hbm, idx_v, rows_v, sem):
    wid = lax.axis_index("s") * NC + lax.axis_index("c")
    base = wid * EPW

    for idx_hbm, out_hbm in ((ii_hbm, xi_hbm), (ij_hbm, xj_hbm)):
        pltpu.sync_copy(idx_hbm.at[pl.ds(base, EPW)], idx_v)

        def fire(j, carry):
            pltpu.async_copy(
                x_hbm.at[idx_v.at[pl.ds(j * CH, CH)]],
                rows_v.at[pl.ds(j * CH, CH)],
                sem,
            )
            return carry

        lax.fori_loop(0, NFULL, fire, 0)
        pltpu.async_copy(
            x_hbm.at[idx_v.at[pl.ds(NFULL * CH, TAIL)]],
            rows_v.at[pl.ds(NFULL * CH, TAIL)],
            sem,
        ).wait()

        def drain(j, carry):
            pltpu.make_async_copy(
                x_hbm.at[pl.ds(0, CH)], rows_v.at[pl.ds(0, CH)], sem
            ).wait()
            return carry

        lax.fori_loop(0, NFULL, drain, 0)
        pltpu.sync_copy(rows_v, out_hbm.at[pl.ds(base, EPW)])


@functools.cache
def _gather_k():
    return pl.kernel(
        _gather_body,
        out_type=(
            jax.ShapeDtypeStruct((E, DN), jnp.float32),
            jax.ShapeDtypeStruct((E, DN), jnp.float32),
        ),
        mesh=_mesh(),
        scratch_types=[
            pltpu.VMEM((EPW,), jnp.int32),
            pltpu.VMEM((EPW, DN), jnp.float32),
            pltpu.SemaphoreType.DMA,
        ],
        compiler_params=pltpu.CompilerParams(use_tc_tiling_on_sc=False),
    )


def _scatter_body(eu_hbm, ii_hbm, zero_hbm, out_hbm, idx_v, upd_v, agg_sh, sem):
    c = lax.axis_index("c")
    s = lax.axis_index("s")
    wid = s * NC + c
    base = wid * EPW

    @pl.when(s == 0)
    def _zero():
        pltpu.sync_copy(zero_hbm, agg_sh)

    def fire(j, carry):
        pltpu.async_copy(
            ii_hbm.at[pl.ds(base + j * CH, CH)], idx_v.at[j], sem
        )
        return carry

    lax.fori_loop(0, NFULL, fire, 0)

    def drain(j, carry):
        pltpu.make_async_copy(
            ii_hbm.at[pl.ds(base, CH)], idx_v.at[0], sem
        ).wait()
        return carry

    lax.fori_loop(0, NFULL, drain, 0)
    pltpu.sync_copy(ii_hbm.at[pl.ds(base, CH)], idx_v.at[NFULL])
    pltpu.sync_copy(
        ii_hbm.at[pl.ds(base + NFULL * CH, TAIL)],
        idx_v.at[NFULL, pl.ds(0, TAIL)],
    )
    pltpu.sync_copy(eu_hbm.at[pl.ds(base, EPW)], upd_v.at[pl.ds(0, EPW)])
    pltpu.sync_copy(
        zero_hbm.at[pl.ds(0, EBUF - EPW)], upd_v.at[pl.ds(EPW, EBUF - EPW)]
    )
    plsc.subcore_barrier()

    def body(j, carry):
        pltpu.sync_copy(
            upd_v.at[pl.ds(j * CH, CH)], agg_sh.at[idx_v.at[j]], add=True
        )
        return carry

    lax.fori_loop(0, NCH, body, 0)
    plsc.subcore_barrier()
    rows = N // NS
    pltpu.sync_copy(
        agg_sh.at[pl.ds(s * rows, rows)], out_hbm.at[c, pl.ds(s * rows, rows)]
    )


@functools.cache
def _scatter_k():
    return pl.kernel(
        _scatter_body,
        out_type=jax.ShapeDtypeStruct((NC, N, DN), jnp.float32),
        mesh=_mesh(),
        scratch_types=[
            pltpu.VMEM((NCH, CH), jnp.int32),
            pltpu.VMEM((EBUF, DN), jnp.float32),
            pltpu.VMEM_SHARED((N, DN), jnp.float32),
            pltpu.SemaphoreType.DMA,
        ],
        compiler_params=pltpu.CompilerParams(use_tc_tiling_on_sc=False),
    )


@functools.cache
def _sel_r():
    r = np.zeros((HID, HID * DN), np.float32)
    for m in range(HID):
        r[m, m * DN : (m + 1) * DN] = 1.0
    return jnp.asarray(r)


@functools.cache
def _sel_s():
    s = np.zeros((HID * DN, DN), np.float32)
    for m in range(HID):
        for k in range(DN):
            s[m * DN + k, k] = 1.0
    return jnp.asarray(s)


def _dense_body(xi_ref, xj_ref, eft_ref, elet_ref, sht_ref,
                wci_ref, wcj_ref, wcf_ref, w1_ref, r_ref, s_ref, out_ref):
    ele = elet_ref[...].T
    ef = eft_ref[...].T
    sh = sht_ref[...].T
    pre = jnp.dot(ele, w1_ref[...], preferred_element_type=jnp.float32)
    h = pre * jax.nn.sigmoid(pre) * sh
    y = (
        jnp.dot(xi_ref[...], wci_ref[...], preferred_element_type=jnp.float32)
        + jnp.dot(xj_ref[...], wcj_ref[...], preferred_element_type=jnp.float32)
        + jnp.dot(ef, wcf_ref[...], preferred_element_type=jnp.float32)
    )
    hx = jnp.dot(h, r_ref[...], preferred_element_type=jnp.float32)
    out_ref[...] = jnp.dot(y * hx, s_ref[...], preferred_element_type=jnp.float32)


def _dense(xi, xj, eft, elet, sht, wci, wcj, wcf, w1):
    grid = E // BE
    full = lambda shape: pl.BlockSpec(shape, lambda i: (0, 0))
    return pl.pallas_call(
        _dense_body,
        grid=(grid,),
        in_specs=[
            pl.BlockSpec((BE, DN), lambda i: (i, 0)),
            pl.BlockSpec((BE, DN), lambda i: (i, 0)),
            pl.BlockSpec((DE, BE), lambda i: (0, i)),
            pl.BlockSpec((FC_IN, BE), lambda i: (0, i)),
            pl.BlockSpec((1, BE), lambda i: (0, i)),
            full((DN, HID * DN)),
            full((DN, HID * DN)),
            full((DE, HID * DN)),
            full((FC_IN, HID)),
            full((HID, HID * DN)),
            full((HID * DN, DN)),
        ],
        out_specs=pl.BlockSpec((BE, DN), lambda i: (i, 0)),
        out_shape=jax.ShapeDtypeStruct((E, DN), jnp.float32),
    )(xi, xj, eft, elet, sht, wci, wcj, wcf, w1, _sel_r(), _sel_s())


def _final_body(p_ref, w_ref, out_ref):
    agg = p_ref[0] + p_ref[1]
    z = jnp.dot(agg, w_ref[...], preferred_element_type=jnp.float32)
    out_ref[...] = z * jax.nn.sigmoid(z)


def _final(parts, w_post):
    return pl.pallas_call(
        _final_body,
        out_shape=jax.ShapeDtypeStruct((N, DN), jnp.float32),
    )(parts, w_post)


def kernel(node_fea, edge_sh, edge_fea, edge_length_embedded, edge_index,
           W_pre, W_fc1, W_fc2, W_post):
    ii = edge_index[0]
    ij = edge_index[1]

    w1n = W_fc1 / np.sqrt(FC_IN)
    wpn = W_pre / np.sqrt(DN)
    wcat = (
        (W_fc2 / np.sqrt(HID))
        .reshape(HID, IN1, DN)
        .transpose(1, 0, 2)
        .reshape(IN1, HID * DN)
    ) / np.sqrt(IN1 * 1)
    wci = wpn @ wcat[0:DN]
    wcj = wpn @ wcat[DN : 2 * DN]
    wcf = wcat[2 * DN :]
    wpostn = W_post / np.sqrt(DN)

    zeros = jnp.zeros((N, DN), jnp.float32)

    xi_raw, xj_raw = _gather_k()(node_fea, ii, ij)
    eu = _dense(xi_raw, xj_raw, edge_fea.T, edge_length_embedded.T, edge_sh.T,
                wci, wcj, wcf, w1n)
    parts = _scatter_k()(eu, ii, zeros)
    return _final(parts, wpostn)
